# baseline (speedup 1.0000x reference)
import jax
import jax.numpy as jnp
from jax import lax
from jax.experimental import pallas as pl
from jax.experimental.pallas import tpu as pltpu
from jax.experimental.pallas import tpu_sc as plsc

K = 65536
DIM = 128
NB = 16384
PTR = 60000
SZ = 0

LEN_A = K - PTR
LEN_B = NB - LEN_A
MID_LO = LEN_B
MID_LEN = PTR - LEN_B

NC, NS = 2, 16
NW = NC * NS

M_WORKERS = 24
M_CHUNK = MID_LEN // M_WORKERS
B_WORKERS = 6
B_CHUNK = LEN_B // B_WORKERS
A_WORKERS = 2
A_CHUNK = LEN_A // A_WORKERS


def _enqueue_body(z_new, t_new, e_new, b_new, z_buf, t_buf, e_buf, b_buf,
                  z_out, t_out, e_out, b_out, f32_v, i32_v):
    c = lax.axis_index("c")
    s = lax.axis_index("s")
    w = s * NC + c

    @pl.when(w < M_WORKERS)
    def _():
        off = MID_LO + w * M_CHUNK
        pltpu.sync_copy(z_buf.at[pl.ds(off, M_CHUNK)],
                        z_out.at[pl.ds(off, M_CHUNK)])

    @pl.when(jnp.logical_and(w >= M_WORKERS, w < M_WORKERS + B_WORKERS))
    def _():
        off = (w - M_WORKERS) * B_CHUNK
        pltpu.sync_copy(z_new.at[pl.ds(LEN_A + off, B_CHUNK)],
                        z_out.at[pl.ds(off, B_CHUNK)])

    @pl.when(w >= M_WORKERS + B_WORKERS)
    def _():
        off = (w - (M_WORKERS + B_WORKERS)) * A_CHUNK
        pltpu.sync_copy(z_new.at[pl.ds(off, A_CHUNK)],
                        z_out.at[pl.ds(PTR + off, A_CHUNK)])

    for wid, new_r, buf_r, out_r, v in ((24, t_new, t_buf, t_out, f32_v),
                                        (25, e_new, e_buf, e_out, f32_v),
                                        (26, b_new, b_buf, b_out, i32_v)):
        @pl.when(w == wid)
        def _(new_r=new_r, buf_r=buf_r, out_r=out_r, v=v):
            for src, src_off, dst_off, ln in (
                    (buf_r, MID_LO, MID_LO, MID_LEN),
                    (new_r, LEN_A, 0, LEN_B),
                    (new_r, 0, PTR, LEN_A)):
                pltpu.sync_copy(src.at[pl.ds(src_off, ln)], v.at[pl.ds(0, ln)])
                pltpu.sync_copy(v.at[pl.ds(0, ln)], out_r.at[pl.ds(dst_off, ln)])


def kernel(z_new, t_new, e_new, b_new, z_buf, t_buf, e_buf, b_buf):
    mesh = plsc.VectorSubcoreMesh(core_axis_name="c", subcore_axis_name="s",
                                  num_cores=NC, num_subcores=NS)
    out_type = (
        jax.ShapeDtypeStruct((K, DIM), jnp.float32),
        jax.ShapeDtypeStruct((K,), jnp.float32),
        jax.ShapeDtypeStruct((K,), jnp.float32),
        jax.ShapeDtypeStruct((K,), jnp.int32),
    )
    scratch = (pltpu.VMEM((MID_LEN,), jnp.float32),
               pltpu.VMEM((MID_LEN,), jnp.int32))
    z, t, e, b = pl.kernel(_enqueue_body, out_type=out_type, mesh=mesh,
                           scratch_types=scratch)(
        z_new, t_new, e_new, b_new, z_buf, t_buf, e_buf, b_buf)
    new_ptr = jnp.asarray((PTR + NB) % K, dtype=jnp.int32)
    new_size = jnp.asarray(min(SZ + NB, K), dtype=jnp.int32)
    return (z, t, e, b, new_ptr, new_size)


# speedup vs baseline: 17.2112x; 17.2112x over previous
import jax
import jax.numpy as jnp
from jax import lax
from jax.experimental import pallas as pl
from jax.experimental.pallas import tpu as pltpu
from jax.experimental.pallas import tpu_sc as plsc

K = 65536
DIM = 128
NB = 16384
PTR = 60000
SZ = 0

LEN_A = K - PTR
LEN_B = NB - LEN_A
MID_LO = LEN_B
MID_LEN = PTR - LEN_B

NC, NS = 2, 16
NW = NC * NS
ROWS_W = K // NW
CHUNK = 256

SCRATCH_1D = 32768


def _segments(w):
    lo, hi = w * ROWS_W, (w + 1) * ROWS_W
    segs = []
    for r_lo, r_hi, src, s_base in ((0, LEN_B, "new", LEN_A),
                                    (MID_LO, PTR, "buf", 0),
                                    (PTR, K, "new", -PTR)):
        a, b = max(lo, r_lo), min(hi, r_hi)
        if a < b:
            segs.append((a, src, a + s_base if src == "new" else a, b - a))
    return segs


def _chunks(segs):
    out = []
    for dst_off, src, src_off, n in segs:
        done = 0
        while done < n:
            ln = min(CHUNK, n - done)
            out.append((dst_off + done, src, src_off + done, ln))
            done += ln
    return out


_TASKS_1D = []
for _bi in range(3):
    _TASKS_1D += [
        (_bi, False, MID_LO, MID_LO, SCRATCH_1D),
        (_bi, False, MID_LO + SCRATCH_1D, MID_LO + SCRATCH_1D,
         MID_LEN - SCRATCH_1D),
        (_bi, True, LEN_A, 0, LEN_B),
        (_bi, True, 0, PTR, LEN_A),
    ]
_TASK_WORKERS = {6 + i: t for i, t in enumerate(_TASKS_1D)}


def _enqueue_body(z_new, t_new, e_new, bf_new, z_buf, t_buf, e_buf, bf_buf,
                  z_out, t_out, e_out, bf_out,
                  v0, v1, v1d, gsem, ssem0, ssem1):
    news = (t_new, e_new, bf_new)
    olds = (t_buf, e_buf, bf_buf)
    outs = (t_out, e_out, bf_out)
    c = lax.axis_index("c")
    s = lax.axis_index("s")
    w = s * NC + c
    bufs = (v0, v1)
    ssems = (ssem0, ssem1)

    def run_worker(wid):
        chunks = _chunks(_segments(wid))
        scatters = [None] * len(chunks)
        for i, (dst_off, src, src_off, ln) in enumerate(chunks):
            b = i % 2
            if i >= 2:
                scatters[i - 2].wait()
            src_ref = z_new if src == "new" else z_buf
            pltpu.async_copy(src_ref.at[pl.ds(src_off, ln)],
                             bufs[b].at[pl.ds(0, ln)], gsem).wait()
            scatters[i] = pltpu.async_copy(bufs[b].at[pl.ds(0, ln)],
                                           z_out.at[pl.ds(dst_off, ln)],
                                           ssems[b])
        for cp in scatters[-2:]:
            if cp is not None:
                cp.wait()
        task = _TASK_WORKERS.get(wid)
        if task is not None:
            bi, from_new, src_off, dst_off, ln = task
            src = news[bi] if from_new else olds[bi]
            pltpu.sync_copy(src.at[pl.ds(src_off, ln)], v1d.at[pl.ds(0, ln)])
            pltpu.sync_copy(v1d.at[pl.ds(0, ln)],
                            outs[bi].at[pl.ds(dst_off, ln)])

    for wid in range(NW):
        @pl.when(w == wid)
        def _(wid=wid):
            run_worker(wid)


def kernel(z_new, t_new, e_new, b_new, z_buf, t_buf, e_buf, b_buf):
    f32 = jnp.float32
    bf_new = jax.lax.bitcast_convert_type(b_new, f32)
    bf_buf = jax.lax.bitcast_convert_type(b_buf, f32)

    mesh = plsc.VectorSubcoreMesh(core_axis_name="c", subcore_axis_name="s",
                                  num_cores=NC, num_subcores=NS)
    out_type = (
        jax.ShapeDtypeStruct((K, DIM), f32),
        jax.ShapeDtypeStruct((K,), f32),
        jax.ShapeDtypeStruct((K,), f32),
        jax.ShapeDtypeStruct((K,), f32),
    )
    scratch = (
        pltpu.VMEM((CHUNK, DIM), f32),
        pltpu.VMEM((CHUNK, DIM), f32),
        pltpu.VMEM((SCRATCH_1D,), f32),
        pltpu.SemaphoreType.DMA,
        pltpu.SemaphoreType.DMA,
        pltpu.SemaphoreType.DMA,
    )
    z, t, e, bf = pl.kernel(_enqueue_body, out_type=out_type, mesh=mesh,
                            scratch_types=scratch)(
        z_new, t_new, e_new, bf_new, z_buf, t_buf, e_buf, bf_buf)
    b = jax.lax.bitcast_convert_type(bf, jnp.int32)
    new_ptr = jnp.asarray((PTR + NB) % K, dtype=jnp.int32)
    new_size = jnp.asarray(min(SZ + NB, K), dtype=jnp.int32)
    return (z, t, e, b, new_ptr, new_size)


# speedup vs baseline: 19.2480x; 1.1183x over previous
import jax
import jax.numpy as jnp
from jax import lax
from jax.experimental import pallas as pl
from jax.experimental.pallas import tpu as pltpu
from jax.experimental.pallas import tpu_sc as plsc

K = 65536
DIM = 128
NB = 16384
PTR = 60000
SZ = 0

LEN_A = K - PTR
LEN_B = NB - LEN_A
MID_LO = LEN_B
MID_LEN = PTR - LEN_B

NC, NS = 2, 16
NW = NC * NS

ZB_LO = 0
ZB_LEN = LEN_B * DIM
ZM_LO = MID_LO * DIM
ZM_LEN = MID_LEN * DIM
ZA_LO = PTR * DIM
ZA_LEN = LEN_A * DIM

ZB_W = ZB_LEN // NW
ZM_W = ZM_LEN // NW
ZA_W = ZA_LEN // NW
ZM_CHUNKS = 4
ZM_CHUNK = ZM_W // ZM_CHUNKS

SM_W = MID_LEN // NW
B_1D_WORKERS = 12
SB_W = LEN_B // B_1D_WORKERS
A_1D_WORKERS = 4
SA_W = LEN_A // A_1D_WORKERS

VBUF = ZM_CHUNK


def _enqueue_body(z_new, t_new, e_new, bf_new, z_buf, t_buf, e_buf, bf_buf,
                  z_out, t_out, e_out, bf_out,
                  v0, v1, gsem, ssem0, ssem1):
    c = lax.axis_index("c")
    s = lax.axis_index("s")
    w = s * NC + c
    bufs = (v0, v1)
    ssems = (ssem0, ssem1)
    news = (t_new, e_new, bf_new)
    olds = (t_buf, e_buf, bf_buf)
    outs = (t_out, e_out, bf_out)

    chunks = [(ZB_LO + w * ZB_W, z_new, LEN_A * DIM + w * ZB_W, ZB_W)]
    for i in range(ZM_CHUNKS):
        o = ZM_LO + w * ZM_W + i * ZM_CHUNK
        chunks.append((o, z_buf, o, ZM_CHUNK))
    chunks.append((ZA_LO + w * ZA_W, z_new, w * ZA_W, ZA_W))
    for bi in range(3):
        o = MID_LO + w * SM_W
        chunks.append(("1d", bi, olds[bi], o, outs[bi], o, SM_W))

    scatters = [None] * len(chunks)
    for i, ch in enumerate(chunks):
        b = i % 2
        if i >= 2:
            scatters[i - 2].wait()
        if ch[0] == "1d":
            _, bi, src, src_off, dst, dst_off, ln = ch
        else:
            dst_off, src, src_off, ln = ch
            dst = z_out
        pltpu.async_copy(src.at[pl.ds(src_off, ln)],
                         bufs[b].at[pl.ds(0, ln)], gsem).wait()
        scatters[i] = pltpu.async_copy(bufs[b].at[pl.ds(0, ln)],
                                       dst.at[pl.ds(dst_off, ln)], ssems[b])
    for cp in scatters[-2:]:
        cp.wait()

    @pl.when(w < B_1D_WORKERS)
    def _():
        for bi in range(3):
            so = LEN_A + w * SB_W
            pltpu.sync_copy(news[bi].at[pl.ds(so, SB_W)],
                            v0.at[pl.ds(0, SB_W)])
            pltpu.sync_copy(v0.at[pl.ds(0, SB_W)],
                            outs[bi].at[pl.ds(w * SB_W, SB_W)])

    @pl.when(jnp.logical_and(w >= B_1D_WORKERS,
                             w < B_1D_WORKERS + A_1D_WORKERS))
    def _():
        k = w - B_1D_WORKERS
        for bi in range(3):
            pltpu.sync_copy(news[bi].at[pl.ds(k * SA_W, SA_W)],
                            v0.at[pl.ds(0, SA_W)])
            pltpu.sync_copy(v0.at[pl.ds(0, SA_W)],
                            outs[bi].at[pl.ds(PTR + k * SA_W, SA_W)])


def kernel(z_new, t_new, e_new, b_new, z_buf, t_buf, e_buf, b_buf):
    f32 = jnp.float32
    zf_new = z_new.reshape(-1)
    zf_buf = z_buf.reshape(-1)
    bf_new = jax.lax.bitcast_convert_type(b_new, f32)
    bf_buf = jax.lax.bitcast_convert_type(b_buf, f32)

    mesh = plsc.VectorSubcoreMesh(core_axis_name="c", subcore_axis_name="s",
                                  num_cores=NC, num_subcores=NS)
    out_type = (
        jax.ShapeDtypeStruct((K * DIM,), f32),
        jax.ShapeDtypeStruct((K,), f32),
        jax.ShapeDtypeStruct((K,), f32),
        jax.ShapeDtypeStruct((K,), f32),
    )
    scratch = (
        pltpu.VMEM((VBUF,), f32),
        pltpu.VMEM((VBUF,), f32),
        pltpu.SemaphoreType.DMA,
        pltpu.SemaphoreType.DMA,
        pltpu.SemaphoreType.DMA,
    )
    zf, t, e, bf = pl.kernel(_enqueue_body, out_type=out_type, mesh=mesh,
                             scratch_types=scratch)(
        zf_new, t_new, e_new, bf_new, zf_buf, t_buf, e_buf, bf_buf)
    z = zf.reshape(K, DIM)
    b = jax.lax.bitcast_convert_type(bf, jnp.int32)
    new_ptr = jnp.asarray((PTR + NB) % K, dtype=jnp.int32)
    new_size = jnp.asarray(min(SZ + NB, K), dtype=jnp.int32)
    return (z, t, e, b, new_ptr, new_size)
